# baseline (speedup 1.0000x reference)
import functools
import math

import jax
import jax.numpy as jnp
from jax import lax
from jax.experimental import pallas as pl
from jax.experimental.pallas import tpu as pltpu
from jax.experimental.pallas import tpu_sc as plsc

B = 1024
L = 200
E = 64
N_IDS = B * L
NW = 32
IDS_PER_W = N_IDS // NW
CHUNK = 128
NCHUNK = IDS_PER_W // CHUNK


def _gather_body(table_hbm, idx_hbm, out_hbm, idx_v, rows_v, sem):
    wid = lax.axis_index("s") * 2 + lax.axis_index("c")
    pltpu.sync_copy(idx_hbm.at[wid], idx_v)
    base = wid * IDS_PER_W

    def body(j, carry):
        pltpu.async_copy(table_hbm.at[idx_v.at[j]], rows_v, sem).wait()
        pltpu.sync_copy(rows_v, out_hbm.at[pl.ds(base + j * CHUNK, CHUNK)])
        return carry

    lax.fori_loop(0, NCHUNK, body, 0)


def _sc_gather(table, idx3):
    mesh = plsc.VectorSubcoreMesh(core_axis_name="c", subcore_axis_name="s")
    return pl.kernel(
        _gather_body,
        mesh=mesh,
        out_type=jax.ShapeDtypeStruct((N_IDS, E), jnp.float32),
        scratch_types=[
            pltpu.VMEM((NCHUNK, CHUNK), jnp.int32),
            pltpu.VMEM((CHUNK, E), jnp.float32),
            pltpu.SemaphoreType.DMA,
        ],
        compiler_params=pltpu.CompilerParams(use_tc_tiling_on_sc=False),
    )(table, idx3)


BB = 8


def _post_body(x_ref, w_ref, b_ref, g_ref, beta_ref, o_ref):
    x = x_ref[...]
    w = w_ref[...]
    zero = jnp.zeros((BB, 1, E), jnp.float32)
    x_prev = jnp.concatenate([zero, x[:, :-1, :]], axis=1)
    x_next = jnp.concatenate([x[:, 1:, :], zero], axis=1)
    y = x_prev * w[0] + x * w[1] + x_next * w[2] + b_ref[...][0]
    y = 0.5 * y * (1.0 + lax.erf(y * (1.0 / math.sqrt(2.0))))
    mean = jnp.mean(y, axis=-1, keepdims=True)
    d = y - mean
    var = jnp.mean(d * d, axis=-1, keepdims=True)
    normed = d * lax.rsqrt(var + 1e-5)
    normed = normed * g_ref[...][0] + beta_ref[...][0]
    l2 = jnp.sqrt(jnp.sum(normed * normed, axis=-1, keepdims=True))
    o_ref[...] = normed / jnp.maximum(l2, 1e-12)


def _tc_post(emb, conv_w, conv_b, ln_gamma, ln_beta):
    w = conv_w[:, 0, :].T
    return pl.pallas_call(
        _post_body,
        grid=(B // BB,),
        in_specs=[
            pl.BlockSpec((BB, L, E), lambda i: (i, 0, 0)),
            pl.BlockSpec((3, E), lambda i: (0, 0)),
            pl.BlockSpec((1, E), lambda i: (0, 0)),
            pl.BlockSpec((1, E), lambda i: (0, 0)),
            pl.BlockSpec((1, E), lambda i: (0, 0)),
        ],
        out_specs=pl.BlockSpec((BB, L, E), lambda i: (i, 0, 0)),
        out_shape=jax.ShapeDtypeStruct((B, L, E), jnp.float32),
    )(emb, w, conv_b.reshape(1, E), ln_gamma.reshape(1, E),
      ln_beta.reshape(1, E))


def kernel(input_ids, table, conv_w, conv_b, ln_gamma, ln_beta):
    idx3 = input_ids.astype(jnp.int32).reshape(NW, NCHUNK, CHUNK)
    emb = _sc_gather(table, idx3).reshape(B, L, E)
    return _tc_post(emb, conv_w, conv_b, ln_gamma, ln_beta)


# speedup vs baseline: 1.0821x; 1.0821x over previous
import functools
import math

import jax
import jax.numpy as jnp
from jax import lax
from jax.experimental import pallas as pl
from jax.experimental.pallas import tpu as pltpu
from jax.experimental.pallas import tpu_sc as plsc

B = 1024
L = 200
E = 64
N_IDS = B * L
NW = 32
IDS_PER_W = N_IDS // NW
CHUNK = 128
NCHUNK = IDS_PER_W // CHUNK


def _gather_body(table_hbm, idx_hbm, out_hbm, idx_v, rows_v, sem):
    wid = lax.axis_index("s") * 2 + lax.axis_index("c")
    pltpu.sync_copy(idx_hbm.at[wid], idx_v)
    base = wid * IDS_PER_W

    def body(j, carry):
        pltpu.async_copy(table_hbm.at[idx_v.at[j]], rows_v, sem).wait()
        pltpu.sync_copy(
            rows_v,
            out_hbm.at[pl.ds(base + j * CHUNK, CHUNK), pl.ds(0, E)],
        )
        return carry

    lax.fori_loop(0, NCHUNK, body, 0)


def _sc_gather(table, idx3):
    mesh = plsc.VectorSubcoreMesh(core_axis_name="c", subcore_axis_name="s")
    return pl.kernel(
        _gather_body,
        mesh=mesh,
        out_type=jax.ShapeDtypeStruct((N_IDS, 128), jnp.float32),
        scratch_types=[
            pltpu.VMEM((NCHUNK, CHUNK), jnp.int32),
            pltpu.VMEM((CHUNK, E), jnp.float32),
            pltpu.SemaphoreType.DMA,
        ],
        compiler_params=pltpu.CompilerParams(use_tc_tiling_on_sc=False),
    )(table, idx3)


BB = 8


def _post_body(x_ref, w_ref, b_ref, g_ref, beta_ref, o_ref):
    x = x_ref[...][:, :, :E]
    w = w_ref[...]
    zero = jnp.zeros((BB, 1, E), jnp.float32)
    x_prev = jnp.concatenate([zero, x[:, :-1, :]], axis=1)
    x_next = jnp.concatenate([x[:, 1:, :], zero], axis=1)
    y = x_prev * w[0] + x * w[1] + x_next * w[2] + b_ref[...][0]
    y = 0.5 * y * (1.0 + lax.erf(y * (1.0 / math.sqrt(2.0))))
    mean = jnp.mean(y, axis=-1, keepdims=True)
    d = y - mean
    var = jnp.mean(d * d, axis=-1, keepdims=True)
    normed = d * lax.rsqrt(var + 1e-5)
    normed = normed * g_ref[...][0] + beta_ref[...][0]
    l2 = jnp.sqrt(jnp.sum(normed * normed, axis=-1, keepdims=True))
    o_ref[...] = normed / jnp.maximum(l2, 1e-12)


def _tc_post(emb, conv_w, conv_b, ln_gamma, ln_beta):
    w = conv_w[:, 0, :].T
    return pl.pallas_call(
        _post_body,
        grid=(B // BB,),
        in_specs=[
            pl.BlockSpec((BB, L, 128), lambda i: (i, 0, 0)),
            pl.BlockSpec((3, E), lambda i: (0, 0)),
            pl.BlockSpec((1, E), lambda i: (0, 0)),
            pl.BlockSpec((1, E), lambda i: (0, 0)),
            pl.BlockSpec((1, E), lambda i: (0, 0)),
        ],
        out_specs=pl.BlockSpec((BB, L, E), lambda i: (i, 0, 0)),
        out_shape=jax.ShapeDtypeStruct((B, L, E), jnp.float32),
    )(emb, w, conv_b.reshape(1, E), ln_gamma.reshape(1, E),
      ln_beta.reshape(1, E))


def kernel(input_ids, table, conv_w, conv_b, ln_gamma, ln_beta):
    idx3 = input_ids.astype(jnp.int32).reshape(NW, NCHUNK, CHUNK)
    emb = _sc_gather(table, idx3).reshape(B, L, 128)
    return _tc_post(emb, conv_w, conv_b, ln_gamma, ln_beta)
